# Initial kernel scaffold; baseline (speedup 1.0000x reference)
import jax
import jax.numpy as jnp
from jax.experimental import pallas as pl


def kernel(index, table):
    raise NotImplementedError("write your pallas kernel here")


# speedup vs baseline: 3.3056x; 3.3056x over previous
import functools

import jax
import jax.numpy as jnp
from jax import lax
from jax.experimental import pallas as pl
from jax.experimental.pallas import tpu as pltpu
from jax.experimental.pallas import tpu_sc as plsc

_NC = 2
_NS = 16
_NW = _NC * _NS
_C = 128


@functools.lru_cache(maxsize=None)
def _make_gather(n_chunks: int, dim: int):
    mesh = plsc.VectorSubcoreMesh(core_axis_name="c", subcore_axis_name="s")
    n_rows = _NW * n_chunks * _C

    @functools.partial(
        pl.kernel,
        mesh=mesh,
        out_type=jax.ShapeDtypeStruct((n_rows, dim), jnp.float32),
        scratch_types=[
            pltpu.VMEM((n_chunks, _C), jnp.int32),
            pltpu.VMEM((_C, dim), jnp.float32),
            pltpu.SemaphoreType.DMA,
        ],
    )
    def k(table_hbm, idx_hbm, out_hbm, idx_v, rows_v, sem):
        wid = lax.axis_index("s") * _NC + lax.axis_index("c")
        base = wid * (n_chunks * _C)
        pltpu.sync_copy(idx_hbm.at[wid], idx_v)

        def body(j, carry):
            pltpu.async_copy(table_hbm.at[idx_v.at[j]], rows_v, sem).wait()
            pltpu.sync_copy(rows_v, out_hbm.at[pl.ds(base + j * _C, _C)])
            return carry

        lax.fori_loop(0, n_chunks, body, 0)

    return k


def kernel(index, table):
    b, l = index.shape
    dim = table.shape[1]
    n = b * l
    assert n % (_NW * _C) == 0
    n_chunks = n // (_NW * _C)
    idx = index.reshape(_NW, n_chunks, _C)
    out = _make_gather(n_chunks, dim)(table, idx)
    return out.reshape(b, l, dim)


# speedup vs baseline: 3.5333x; 1.0689x over previous
import functools

import jax
import jax.numpy as jnp
from jax import lax
from jax.experimental import pallas as pl
from jax.experimental.pallas import tpu as pltpu
from jax.experimental.pallas import tpu_sc as plsc

_NC = 2
_NS = 16
_NW = _NC * _NS
_C = 128


@functools.lru_cache(maxsize=None)
def _make_gather(n_chunks: int, dim: int):
    mesh = plsc.VectorSubcoreMesh(core_axis_name="c", subcore_axis_name="s")
    n_rows = _NW * n_chunks * _C

    @functools.partial(
        pl.kernel,
        mesh=mesh,
        out_type=jax.ShapeDtypeStruct((n_rows, dim), jnp.float32),
        scratch_types=[
            pltpu.VMEM((n_chunks, _C), jnp.int32),
            pltpu.VMEM((2, _C, dim), jnp.float32),
            pltpu.SemaphoreType.DMA((2,)),
            pltpu.SemaphoreType.DMA((2,)),
        ],
    )
    def k(table_hbm, idx_hbm, out_hbm, idx_v, rows_v, gsem, wsem):
        wid = lax.axis_index("s") * _NC + lax.axis_index("c")
        base = wid * (n_chunks * _C)
        pltpu.sync_copy(idx_hbm.at[wid], idx_v)

        pltpu.async_copy(table_hbm.at[idx_v.at[0]], rows_v.at[0], gsem.at[0])

        def body(j, carry):
            b = lax.rem(j, 2)
            nb = 1 - b
            pltpu.make_async_copy(
                table_hbm.at[idx_v.at[j]], rows_v.at[b], gsem.at[b]
            ).wait()

            @pl.when(j >= 1)
            def _():
                pltpu.make_async_copy(
                    rows_v.at[nb],
                    out_hbm.at[pl.ds(base + (j - 1) * _C, _C)],
                    wsem.at[nb],
                ).wait()

            @pl.when(j + 1 < n_chunks)
            def _():
                pltpu.async_copy(
                    table_hbm.at[idx_v.at[j + 1]], rows_v.at[nb], gsem.at[nb]
                )

            pltpu.async_copy(
                rows_v.at[b], out_hbm.at[pl.ds(base + j * _C, _C)], wsem.at[b]
            )
            return carry

        lax.fori_loop(0, n_chunks, body, 0)

        lb = (n_chunks - 1) % 2
        pltpu.make_async_copy(
            rows_v.at[lb],
            out_hbm.at[pl.ds(base + (n_chunks - 1) * _C, _C)],
            wsem.at[lb],
        ).wait()

    return k


def kernel(index, table):
    b, l = index.shape
    dim = table.shape[1]
    n = b * l
    assert n % (_NW * _C) == 0
    n_chunks = n // (_NW * _C)
    idx = index.reshape(_NW, n_chunks, _C)
    out = _make_gather(n_chunks, dim)(table, idx)
    return out.reshape(b, l, dim)


# speedup vs baseline: 3.8418x; 1.0873x over previous
import functools

import jax
import jax.numpy as jnp
from jax import lax
from jax.experimental import pallas as pl
from jax.experimental.pallas import tpu as pltpu
from jax.experimental.pallas import tpu_sc as plsc

_NC = 2
_NS = 16
_NW = _NC * _NS
_C = 128
_NBUF = 4


@functools.lru_cache(maxsize=None)
def _make_gather(n_chunks: int, dim: int):
    mesh = plsc.VectorSubcoreMesh(core_axis_name="c", subcore_axis_name="s")
    n_rows = _NW * n_chunks * _C

    @functools.partial(
        pl.kernel,
        mesh=mesh,
        out_type=jax.ShapeDtypeStruct((n_rows, dim), jnp.float32),
        scratch_types=[
            pltpu.VMEM((n_chunks, _C), jnp.int32),
            pltpu.VMEM((_NBUF, _C, dim), jnp.float32),
            pltpu.SemaphoreType.DMA((_NBUF,)),
            pltpu.SemaphoreType.DMA((_NBUF,)),
        ],
    )
    def k(table_hbm, idx_hbm, out_hbm, idx_v, rows_v, gsem, wsem):
        wid = lax.axis_index("s") * _NC + lax.axis_index("c")
        base = wid * (n_chunks * _C)
        pltpu.sync_copy(idx_hbm.at[wid], idx_v)

        for u in range(_NBUF - 1):
            pltpu.async_copy(table_hbm.at[idx_v.at[u]], rows_v.at[u], gsem.at[u])

        def body(j, carry):
            b = lax.rem(j, _NBUF)
            pltpu.make_async_copy(
                table_hbm.at[idx_v.at[j]], rows_v.at[b], gsem.at[b]
            ).wait()
            pltpu.async_copy(
                rows_v.at[b], out_hbm.at[pl.ds(base + j * _C, _C)], wsem.at[b]
            )

            bn = lax.rem(j + _NBUF - 1, _NBUF)

            @pl.when(j + _NBUF - 1 < n_chunks)
            def _():
                @pl.when(j >= 1)
                def _():
                    pltpu.make_async_copy(
                        rows_v.at[bn],
                        out_hbm.at[pl.ds(base + (j - 1) * _C, _C)],
                        wsem.at[bn],
                    ).wait()

                pltpu.async_copy(
                    table_hbm.at[idx_v.at[j + _NBUF - 1]],
                    rows_v.at[bn],
                    gsem.at[bn],
                )

            return carry

        lax.fori_loop(0, n_chunks, body, 0)

        for t in range(_NBUF):
            j = n_chunks - _NBUF + t
            pltpu.make_async_copy(
                rows_v.at[j % _NBUF],
                out_hbm.at[pl.ds(base + j * _C, _C)],
                wsem.at[j % _NBUF],
            ).wait()

    return k


def kernel(index, table):
    b, l = index.shape
    dim = table.shape[1]
    n = b * l
    assert n % (_NW * _C) == 0
    n_chunks = n // (_NW * _C)
    idx = index.reshape(_NW, n_chunks, _C)
    out = _make_gather(n_chunks, dim)(table, idx)
    return out.reshape(b, l, dim)


# speedup vs baseline: 6.0450x; 1.5735x over previous
import functools

import jax
import jax.numpy as jnp
from jax import lax
from jax.experimental import pallas as pl
from jax.experimental.pallas import tpu as pltpu
from jax.experimental.pallas import tpu_sc as plsc

_NC = 2
_NS = 16
_NW = _NC * _NS
_C = 128


@functools.lru_cache(maxsize=None)
def _make_gather(b: int, l: int, dim: int):
    mesh = plsc.VectorSubcoreMesh(core_axis_name="c", subcore_axis_name="s")
    rows_per_w = b // _NW
    cq = -(-l // _C)
    offs = [min(q * _C, l - _C) for q in range(cq)]

    @functools.partial(
        pl.kernel,
        mesh=mesh,
        out_type=jax.ShapeDtypeStruct((b, l, dim), jnp.float32),
        scratch_types=[
            pltpu.VMEM((2, cq, _C), jnp.int32),
            pltpu.VMEM((2, l, dim), jnp.float32),
            pltpu.SemaphoreType.DMA((2,)),
            pltpu.SemaphoreType.DMA((2,)),
            pltpu.SemaphoreType.DMA((2,)),
        ],
    )
    def k(table_hbm, idx_hbm, out_hbm, idx_v, rows_v, isem, gsem, wsem):
        wid = lax.axis_index("s") * _NC + lax.axis_index("c")
        row0 = wid * rows_per_w

        for u in range(2):
            pltpu.async_copy(idx_hbm.at[row0 + u], idx_v.at[u], isem.at[u])

        def body(r, carry):
            bb = lax.rem(r, 2)
            pltpu.make_async_copy(
                idx_hbm.at[row0 + r], idx_v.at[bb], isem.at[bb]
            ).wait()

            @pl.when(r >= 2)
            def _():
                pltpu.make_async_copy(
                    rows_v.at[bb], out_hbm.at[row0 + r - 2], wsem.at[bb]
                ).wait()

            for q in range(cq):
                pltpu.async_copy(
                    table_hbm.at[idx_v.at[bb, q]],
                    rows_v.at[bb, pl.ds(offs[q], _C)],
                    gsem.at[bb],
                )
            for q in range(cq):
                pltpu.make_async_copy(
                    table_hbm.at[idx_v.at[bb, q]],
                    rows_v.at[bb, pl.ds(offs[q], _C)],
                    gsem.at[bb],
                ).wait()

            pltpu.async_copy(rows_v.at[bb], out_hbm.at[row0 + r], wsem.at[bb])

            @pl.when(r + 2 < rows_per_w)
            def _():
                pltpu.async_copy(
                    idx_hbm.at[row0 + r + 2], idx_v.at[bb], isem.at[bb]
                )

            return carry

        lax.fori_loop(0, rows_per_w, body, 0)

        for t in range(2):
            r = rows_per_w - 2 + t
            pltpu.make_async_copy(
                rows_v.at[r % 2], out_hbm.at[row0 + r], wsem.at[r % 2]
            ).wait()

    return k


def kernel(index, table):
    b, l = index.shape
    dim = table.shape[1]
    assert b % _NW == 0 and l >= _C
    cq = -(-l // _C)
    offs = [min(q * _C, l - _C) for q in range(cq)]
    idx_c = jnp.stack([lax.slice_in_dim(index, o, o + _C, axis=1) for o in offs],
                      axis=1)
    return _make_gather(b, l, dim)(table, idx_c)


# speedup vs baseline: 6.0485x; 1.0006x over previous
import functools

import jax
import jax.numpy as jnp
from jax import lax
from jax.experimental import pallas as pl
from jax.experimental.pallas import tpu as pltpu
from jax.experimental.pallas import tpu_sc as plsc

_NC = 2
_NS = 16
_NW = _NC * _NS
_C = 128


@functools.lru_cache(maxsize=None)
def _make_gather(b: int, l: int, dim: int):
    mesh = plsc.VectorSubcoreMesh(
        core_axis_name="c", subcore_axis_name="s", num_cores=_NC
    )
    rows_per_w = b // _NW
    cq = -(-l // _C)
    offs = [min(q * _C, l - _C) for q in range(cq)]

    @functools.partial(
        pl.kernel,
        mesh=mesh,
        out_type=jax.ShapeDtypeStruct((b, l, dim), jnp.float32),
        scratch_types=[
            pltpu.VMEM((2, cq, _C), jnp.int32),
            pltpu.VMEM((2, l, dim), jnp.float32),
            pltpu.SemaphoreType.DMA((2,)),
            pltpu.SemaphoreType.DMA((2,)),
            pltpu.SemaphoreType.DMA((2,)),
        ],
    )
    def k(table_hbm, idx_hbm, out_hbm, idx_v, rows_v, isem, gsem, wsem):
        wid = lax.axis_index("s") * _NC + lax.axis_index("c")
        row0 = wid * rows_per_w

        for u in range(2):
            pltpu.async_copy(idx_hbm.at[row0 + u], idx_v.at[u], isem.at[u])

        def body(r, carry):
            bb = lax.rem(r, 2)
            pltpu.make_async_copy(
                idx_hbm.at[row0 + r], idx_v.at[bb], isem.at[bb]
            ).wait()

            @pl.when(r >= 2)
            def _():
                pltpu.make_async_copy(
                    rows_v.at[bb], out_hbm.at[row0 + r - 2], wsem.at[bb]
                ).wait()

            for q in range(cq):
                pltpu.async_copy(
                    table_hbm.at[idx_v.at[bb, q]],
                    rows_v.at[bb, pl.ds(offs[q], _C)],
                    gsem.at[bb],
                )
            for q in range(cq):
                pltpu.make_async_copy(
                    table_hbm.at[idx_v.at[bb, q]],
                    rows_v.at[bb, pl.ds(offs[q], _C)],
                    gsem.at[bb],
                ).wait()

            pltpu.async_copy(rows_v.at[bb], out_hbm.at[row0 + r], wsem.at[bb])

            @pl.when(r + 2 < rows_per_w)
            def _():
                pltpu.async_copy(
                    idx_hbm.at[row0 + r + 2], idx_v.at[bb], isem.at[bb]
                )

            return carry

        lax.fori_loop(0, rows_per_w, body, 0)

        for t in range(2):
            r = rows_per_w - 2 + t
            pltpu.make_async_copy(
                rows_v.at[r % 2], out_hbm.at[row0 + r], wsem.at[r % 2]
            ).wait()

    return k


def kernel(index, table):
    b, l = index.shape
    dim = table.shape[1]
    assert b % _NW == 0 and l >= _C
    cq = -(-l // _C)
    offs = [min(q * _C, l - _C) for q in range(cq)]
    idx_c = jnp.stack([lax.slice_in_dim(index, o, o + _C, axis=1) for o in offs],
                      axis=1)
    return _make_gather(b, l, dim)(table, idx_c)


# speedup vs baseline: 11.3455x; 1.8757x over previous
import functools

import jax
import jax.numpy as jnp
from jax import lax
from jax.experimental import pallas as pl
from jax.experimental.pallas import tpu as pltpu
from jax.experimental.pallas import tpu_sc as plsc

_NC = 2
_NS = 16
_NW = _NC * _NS
_C = 128
_NBUF = 7


@functools.lru_cache(maxsize=None)
def _make_gather(b: int, l: int, dim: int):
    mesh = plsc.VectorSubcoreMesh(
        core_axis_name="c", subcore_axis_name="s", num_cores=_NC
    )
    kb = b // _C
    n_chunks = l * kb // _NW

    @functools.partial(
        pl.kernel,
        mesh=mesh,
        out_type=jax.ShapeDtypeStruct((l, b, dim), jnp.float32),
        scratch_types=[
            pltpu.VMEM((n_chunks, _C), jnp.int32),
            pltpu.VMEM((_NBUF, _C, dim), jnp.float32),
            pltpu.SemaphoreType.DMA((_NBUF,)),
            pltpu.SemaphoreType.DMA((_NBUF,)),
        ],
    )
    def k(table_hbm, idx_hbm, out_hbm, idx_v, rows_v, gsem, wsem):
        wid = lax.axis_index("s") * _NC + lax.axis_index("c")
        base = wid * n_chunks
        pltpu.sync_copy(idx_hbm.at[wid], idx_v)

        def chunk_out(j):
            c = base + j
            return out_hbm.at[lax.div(c, kb), pl.ds(lax.rem(c, kb) * _C, _C)]

        for u in range(_NBUF - 1):
            pltpu.async_copy(table_hbm.at[idx_v.at[u]], rows_v.at[u], gsem.at[u])

        def body(j, carry):
            bb = lax.rem(j, _NBUF)
            pltpu.make_async_copy(
                table_hbm.at[idx_v.at[j]], rows_v.at[bb], gsem.at[bb]
            ).wait()
            pltpu.async_copy(rows_v.at[bb], chunk_out(j), wsem.at[bb])

            bn = lax.rem(j + _NBUF - 1, _NBUF)

            @pl.when(j + _NBUF - 1 < n_chunks)
            def _():
                @pl.when(j >= 1)
                def _():
                    pltpu.make_async_copy(
                        rows_v.at[bn], chunk_out(j - 1), wsem.at[bn]
                    ).wait()

                pltpu.async_copy(
                    table_hbm.at[idx_v.at[j + _NBUF - 1]],
                    rows_v.at[bn],
                    gsem.at[bn],
                )

            return carry

        lax.fori_loop(0, n_chunks, body, 0)

        for t in range(_NBUF):
            j = n_chunks - _NBUF + t
            pltpu.make_async_copy(
                rows_v.at[j % _NBUF], chunk_out(j), wsem.at[j % _NBUF]
            ).wait()

    return k


def kernel(index, table):
    b, l = index.shape
    dim = table.shape[1]
    assert b % _C == 0 and (l * b // _C) % _NW == 0
    idx_c = jnp.transpose(index).reshape(_NW, l * b // _C // _NW, _C)
    out_p = _make_gather(b, l, dim)(table, idx_c)
    return jnp.transpose(out_p, (1, 0, 2))
